# baseline (speedup 1.0000x reference)
import functools

import jax
import jax.numpy as jnp
from jax.experimental import pallas as pl
from jax.experimental.pallas import tpu as pltpu

_B = 16384
_D = 128
_NUM_SPLITS = 16
_N_HEADS = 8
_ACTION_DIM = _NUM_SPLITS + 1
_NH = _N_HEADS * _ACTION_DIM
_TB = 1024


def _body(x_ref, wt_ref, b_ref, g_ref, act_ref, lp_ref):
    logits = jnp.dot(x_ref[...], wt_ref[...],
                     preferred_element_type=jnp.float32) + b_ref[...]
    g = g_ref[...]
    tb = logits.shape[0]
    iota_i = jax.lax.broadcasted_iota(jnp.int32, (tb, _ACTION_DIM), 1)
    iota = iota_i.astype(jnp.float32)
    taken = jnp.zeros((tb, 1), jnp.float32)
    lp_sum = jnp.zeros((tb, 1), jnp.float32)
    acts = []
    for idx in range(_N_HEADS):
        sl = slice(idx * _ACTION_DIM, (idx + 1) * _ACTION_DIM)
        l = logits[:, sl]
        gi = g[:, sl]
        mask = iota <= (jnp.float32(_NUM_SPLITS) - taken)
        ml = jnp.where(mask, l, jnp.float32(-1e10))
        y = gi + ml
        m = jnp.max(y, axis=1, keepdims=True)
        a = jnp.min(jnp.where(y == m, iota, jnp.float32(1e9)),
                    axis=1, keepdims=True)
        mm = jnp.max(ml, axis=1, keepdims=True)
        lse = jnp.log(jnp.sum(jnp.exp(ml - mm), axis=1, keepdims=True))
        ml_a = jnp.sum(jnp.where(iota == a, ml - mm, 0.0), axis=1,
                       keepdims=True)
        lp_sum = lp_sum + (ml_a - lse)
        taken = taken + a
        acts.append(a)
    act_ref[...] = jnp.concatenate(acts, axis=1)
    lp_ref[...] = lp_sum


@functools.partial(jax.jit, static_argnames=())
def kernel(x, W, b):
    sample_key = jax.random.key(12345)
    g = jnp.concatenate(
        [jax.random.gumbel(jax.random.fold_in(sample_key, i),
                           (_B, _ACTION_DIM), jnp.float32)
         for i in range(_N_HEADS)], axis=1)
    wt = W.reshape(_NH, _D).T
    bflat = b.reshape(1, _NH)
    grid = (_B // _TB,)
    actions, lp = pl.pallas_call(
        _body,
        grid=grid,
        in_specs=[
            pl.BlockSpec((_TB, _D), lambda i: (i, 0)),
            pl.BlockSpec((_D, _NH), lambda i: (0, 0)),
            pl.BlockSpec((1, _NH), lambda i: (0, 0)),
            pl.BlockSpec((_TB, _NH), lambda i: (i, 0)),
        ],
        out_specs=[
            pl.BlockSpec((_TB, _N_HEADS), lambda i: (i, 0)),
            pl.BlockSpec((_TB, 1), lambda i: (i, 0)),
        ],
        out_shape=[
            jax.ShapeDtypeStruct((_B, _N_HEADS), jnp.float32),
            jax.ShapeDtypeStruct((_B, 1), jnp.float32),
        ],
        compiler_params=pltpu.CompilerParams(
            dimension_semantics=("arbitrary",),
        ),
    )(x, wt, bflat, g)
    return actions, lp


# speedup vs baseline: 1.2755x; 1.2755x over previous
import functools

import jax
import jax.numpy as jnp
from jax.experimental import pallas as pl
from jax.experimental.pallas import tpu as pltpu

_B = 16384
_D = 128
_NUM_SPLITS = 16
_N_HEADS = 8
_ACTION_DIM = _NUM_SPLITS + 1
_NH = _N_HEADS * _ACTION_DIM
_TB = 1024


def _body(x_ref, wt_ref, b_ref, g_ref, act_ref, lp_ref):
    logits = jnp.dot(x_ref[...], wt_ref[...],
                     preferred_element_type=jnp.float32) + b_ref[...]
    tb = logits.shape[0]
    iota_i = jax.lax.broadcasted_iota(jnp.int32, (tb, _ACTION_DIM), 1)
    iota = iota_i.astype(jnp.float32)
    taken = jnp.zeros((tb, 1), jnp.float32)
    lp_sum = jnp.zeros((tb, 1), jnp.float32)
    acts = []
    for idx in range(_N_HEADS):
        l = logits[:, idx * _ACTION_DIM:(idx + 1) * _ACTION_DIM]
        gi = g_ref[idx]
        mask = iota <= (jnp.float32(_NUM_SPLITS) - taken)
        ml = jnp.where(mask, l, jnp.float32(-1e10))
        y = gi + ml
        m = jnp.max(y, axis=1, keepdims=True)
        a = jnp.min(jnp.where(y == m, iota, jnp.float32(1e9)),
                    axis=1, keepdims=True)
        mm = jnp.max(ml, axis=1, keepdims=True)
        lse = jnp.log(jnp.sum(jnp.exp(ml - mm), axis=1, keepdims=True))
        ml_a = jnp.sum(jnp.where(iota == a, ml - mm, 0.0), axis=1,
                       keepdims=True)
        lp_sum = lp_sum + (ml_a - lse)
        taken = taken + a
        acts.append(a)
    act_ref[...] = jnp.concatenate(acts, axis=1)
    lp_ref[...] = lp_sum


@functools.partial(jax.jit, static_argnames=())
def kernel(x, W, b):
    sample_key = jax.random.key(12345)
    keys = jax.vmap(jax.random.fold_in, (None, 0))(
        sample_key, jnp.arange(_N_HEADS, dtype=jnp.uint32))
    g = jax.vmap(
        lambda k: jax.random.gumbel(k, (_B * _ACTION_DIM,), jnp.float32)
    )(keys).reshape(_N_HEADS, _B, _ACTION_DIM)
    wt = W.reshape(_NH, _D).T
    bflat = b.reshape(1, _NH)
    grid = (_B // _TB,)
    actions, lp = pl.pallas_call(
        _body,
        grid=grid,
        in_specs=[
            pl.BlockSpec((_TB, _D), lambda i: (i, 0)),
            pl.BlockSpec((_D, _NH), lambda i: (0, 0)),
            pl.BlockSpec((1, _NH), lambda i: (0, 0)),
            pl.BlockSpec((_N_HEADS, _TB, _ACTION_DIM), lambda i: (0, i, 0)),
        ],
        out_specs=[
            pl.BlockSpec((_TB, _N_HEADS), lambda i: (i, 0)),
            pl.BlockSpec((_TB, 1), lambda i: (i, 0)),
        ],
        out_shape=[
            jax.ShapeDtypeStruct((_B, _N_HEADS), jnp.float32),
            jax.ShapeDtypeStruct((_B, 1), jnp.float32),
        ],
        compiler_params=pltpu.CompilerParams(
            dimension_semantics=("arbitrary",),
        ),
    )(x, wt, bflat, g)
    return actions, lp


# speedup vs baseline: 5.6928x; 4.4631x over previous
import functools

import jax
import jax.numpy as jnp
from jax.experimental import pallas as pl
from jax.experimental.pallas import tpu as pltpu

_B = 16384
_D = 128
_NUM_SPLITS = 16
_N_HEADS = 8
_ACTION_DIM = _NUM_SPLITS + 1
_NH = _N_HEADS * _ACTION_DIM
_TB = 1024


def _body(xt_ref, w2_ref, b_ref, g_ref, act_ref, lp_ref):
    logits = jnp.dot(w2_ref[...], xt_ref[...],
                     preferred_element_type=jnp.float32) + b_ref[...]
    tb = logits.shape[1]
    iota_i = jax.lax.broadcasted_iota(jnp.int32, (_ACTION_DIM, tb), 0)
    iota = iota_i.astype(jnp.float32)
    taken = jnp.zeros((1, tb), jnp.float32)
    lp_sum = jnp.zeros((1, tb), jnp.float32)
    acts = []
    for idx in range(_N_HEADS):
        l = logits[idx * _ACTION_DIM:(idx + 1) * _ACTION_DIM, :]
        gi = g_ref[idx]
        mask = iota <= (jnp.float32(_NUM_SPLITS) - taken)
        ml = jnp.where(mask, l, jnp.float32(-1e10))
        y = gi + ml
        m = jnp.max(y, axis=0, keepdims=True)
        a = jnp.min(jnp.where(y == m, iota, jnp.float32(1e9)),
                    axis=0, keepdims=True)
        mm = jnp.max(ml, axis=0, keepdims=True)
        lse = jnp.log(jnp.sum(jnp.exp(ml - mm), axis=0, keepdims=True))
        ml_a = jnp.sum(jnp.where(iota == a, ml - mm, 0.0), axis=0,
                       keepdims=True)
        lp_sum = lp_sum + (ml_a - lse)
        taken = taken + a
        acts.append(a)
    act_ref[...] = jnp.concatenate(acts, axis=0)
    lp_ref[...] = lp_sum


@functools.partial(jax.jit, static_argnames=())
def kernel(x, W, b):
    sample_key = jax.random.key(12345)
    keys = jax.vmap(jax.random.fold_in, (None, 0))(
        sample_key, jnp.arange(_N_HEADS, dtype=jnp.uint32))
    g = jax.vmap(
        lambda k: jax.random.gumbel(k, (_B * _ACTION_DIM,), jnp.float32)
    )(keys)
    gt = g.reshape(_N_HEADS, _B, _ACTION_DIM).transpose(0, 2, 1)
    xt = x.T
    w2 = W.reshape(_NH, _D)
    b2 = b.reshape(_NH, 1)
    grid = (_B // _TB,)
    actions_t, lp_t = pl.pallas_call(
        _body,
        grid=grid,
        in_specs=[
            pl.BlockSpec((_D, _TB), lambda i: (0, i)),
            pl.BlockSpec((_NH, _D), lambda i: (0, 0)),
            pl.BlockSpec((_NH, 1), lambda i: (0, 0)),
            pl.BlockSpec((_N_HEADS, _ACTION_DIM, _TB), lambda i: (0, 0, i)),
        ],
        out_specs=[
            pl.BlockSpec((_N_HEADS, _TB), lambda i: (0, i)),
            pl.BlockSpec((1, _TB), lambda i: (0, i)),
        ],
        out_shape=[
            jax.ShapeDtypeStruct((_N_HEADS, _B), jnp.float32),
            jax.ShapeDtypeStruct((1, _B), jnp.float32),
        ],
        compiler_params=pltpu.CompilerParams(
            dimension_semantics=("arbitrary",),
        ),
    )(xt, w2, b2, gt)
    return actions_t.T, lp_t.T


# speedup vs baseline: 6.8614x; 1.2053x over previous
import functools

import jax
import jax.numpy as jnp
from jax.experimental import pallas as pl
from jax.experimental.pallas import tpu as pltpu

_B = 16384
_D = 128
_NUM_SPLITS = 16
_N_HEADS = 8
_ACTION_DIM = _NUM_SPLITS + 1
_NH = _N_HEADS * _ACTION_DIM
_TB = 1024


def _body(x_ref, w2_ref, b_ref, g_ref, act_ref, lp_ref):
    logits = jax.lax.dot_general(
        w2_ref[...], x_ref[...],
        dimension_numbers=(((1,), (1,)), ((), ())),
        preferred_element_type=jnp.float32) + b_ref[...]
    tb = logits.shape[1]
    iota_i = jax.lax.broadcasted_iota(jnp.int32, (_ACTION_DIM, tb), 0)
    iota = iota_i.astype(jnp.float32)
    taken = jnp.zeros((1, tb), jnp.float32)
    lp_sum = jnp.zeros((1, tb), jnp.float32)
    acts = []
    for idx in range(_N_HEADS):
        l = logits[idx * _ACTION_DIM:(idx + 1) * _ACTION_DIM, :]
        gi = g_ref[idx]
        mask = iota <= (jnp.float32(_NUM_SPLITS) - taken)
        ml = jnp.where(mask, l, jnp.float32(-1e10))
        y = gi + ml
        m = jnp.max(y, axis=0, keepdims=True)
        a = jnp.min(jnp.where(y == m, iota, jnp.float32(1e9)),
                    axis=0, keepdims=True)
        mm = jnp.max(ml, axis=0, keepdims=True)
        lse = jnp.log(jnp.sum(jnp.exp(ml - mm), axis=0, keepdims=True))
        ml_a = jnp.sum(jnp.where(iota == a, ml - mm, 0.0), axis=0,
                       keepdims=True)
        lp_sum = lp_sum + (ml_a - lse)
        taken = taken + a
        acts.append(a)
    act_ref[...] = jnp.concatenate(acts, axis=0)
    lp_ref[...] = lp_sum


@functools.partial(jax.jit, static_argnames=())
def kernel(x, W, b):
    sample_key = jax.random.key(12345)
    keys = jax.vmap(jax.random.fold_in, (None, 0))(
        sample_key, jnp.arange(_N_HEADS, dtype=jnp.uint32))
    g = jax.vmap(
        lambda k: jax.random.gumbel(k, (_B * _ACTION_DIM,), jnp.float32)
    )(keys)
    gt = g.reshape(_N_HEADS, _B, _ACTION_DIM).transpose(0, 2, 1)
    w2 = W.reshape(_NH, _D)
    b2 = b.reshape(_NH, 1)
    grid = (_B // _TB,)
    actions_t, lp_t = pl.pallas_call(
        _body,
        grid=grid,
        in_specs=[
            pl.BlockSpec((_TB, _D), lambda i: (i, 0)),
            pl.BlockSpec((_NH, _D), lambda i: (0, 0)),
            pl.BlockSpec((_NH, 1), lambda i: (0, 0)),
            pl.BlockSpec((_N_HEADS, _ACTION_DIM, _TB), lambda i: (0, 0, i)),
        ],
        out_specs=[
            pl.BlockSpec((_N_HEADS, _TB), lambda i: (0, i)),
            pl.BlockSpec((1, _TB), lambda i: (0, i)),
        ],
        out_shape=[
            jax.ShapeDtypeStruct((_N_HEADS, _B), jnp.float32),
            jax.ShapeDtypeStruct((1, _B), jnp.float32),
        ],
        compiler_params=pltpu.CompilerParams(
            dimension_semantics=("arbitrary",),
        ),
    )(x, w2, b2, gt)
    return actions_t.T, lp_t.T
